# Initial kernel scaffold; baseline (speedup 1.0000x reference)
import jax
import jax.numpy as jnp
from jax.experimental import pallas as pl


def kernel(idx, embedding, covariance):
    raise NotImplementedError("write your pallas kernel here")


# speedup vs baseline: 1.2218x; 1.2218x over previous
import functools

import jax
import jax.numpy as jnp
from jax import lax
from jax.experimental import pallas as pl
from jax.experimental.pallas import tpu as pltpu
from jax.experimental.pallas import tpu_sc as plsc

EPS = 1e-7
D = 32
DP1 = 33
DPAD = 40


def _sc_gather(idx_flat, emb_pad, covariance, chunk=1024):
    n = idx_flat.shape[0]
    NC, NS = 2, 16
    NW = NC * NS
    per_w = n // NW
    n_chunks = per_w // chunk
    mesh = plsc.VectorSubcoreMesh(core_axis_name="c", subcore_axis_name="s")

    @functools.partial(
        pl.kernel,
        out_type=(jax.ShapeDtypeStruct((n, DPAD), jnp.float32),
                  jax.ShapeDtypeStruct((n, D), jnp.float32)),
        mesh=mesh,
        compiler_params=pltpu.CompilerParams(use_tc_tiling_on_sc=False),
        scratch_types=[
            pltpu.VMEM((chunk,), jnp.int32),
            pltpu.VMEM((chunk, DPAD), jnp.float32),
            pltpu.VMEM((chunk, D), jnp.float32),
            pltpu.SemaphoreType.DMA,
            pltpu.SemaphoreType.DMA,
        ],
    )
    def gather_kernel(idx_hbm, emb_hbm, cov_hbm, emb_out, cov_out,
                      idx_v, emb_v, cov_v, sem_e, sem_c):
        wid = lax.axis_index("s") * NC + lax.axis_index("c")
        base = wid * per_w
        for k in range(n_chunks):
            start = base + k * chunk
            pltpu.sync_copy(idx_hbm.at[pl.ds(start, chunk)], idx_v)
            ce = pltpu.async_copy(emb_hbm.at[idx_v], emb_v, sem_e)
            cc = pltpu.async_copy(cov_hbm.at[idx_v], cov_v, sem_c)
            ce.wait()
            cc.wait()
            pltpu.sync_copy(emb_v, emb_out.at[pl.ds(start, chunk)])
            pltpu.sync_copy(cov_v, cov_out.at[pl.ds(start, chunk)])

    return gather_kernel(idx_flat, emb_pad, covariance)


def _math_body(e_ref, c_ref, o_ref):
    e = e_ref[...]
    cv = c_ref[...]
    src = e[:, 0:1, :]
    tgt = e[:, 1:, :]
    alpha = -(jnp.sum(src[..., :D] * tgt[..., :D], axis=-1, keepdims=True)
              - src[..., D:DP1] * tgt[..., D:DP1])
    alpha = 1.0 + jnp.maximum(alpha - 1.0, EPS)
    sq = jnp.sqrt(jnp.maximum(alpha * alpha - 1.0, 0.0))
    denom = jnp.maximum(sq, EPS)
    acosh = jnp.log(alpha + sq)
    to_t_head = acosh * (tgt[..., :D] - alpha * src[..., :D]) / denom
    to_t_last = acosh * (tgt[..., D:DP1] - alpha * src[..., D:DP1]) / denom
    beta = src[..., D:DP1]
    w_head = -(beta * src[..., :D])
    w_last = 1.0 - beta * src[..., D:DP1]
    mdot = (jnp.sum(w_head * to_t_head, axis=-1, keepdims=True)
            - w_last * to_t_last)
    scale = mdot / jnp.maximum(beta + 1.0, EPS)
    x = to_t_head + scale * src[..., :D]
    sig = jnp.where(cv > 0, cv, (1.0 - EPS) * (jnp.exp(cv) - 1.0)) + 1.0
    sig = jnp.maximum(sig, EPS)
    s0 = sig[:, 0:1, :]
    st = sig[:, 1:, :]
    trace = jnp.sum(st / s0, axis=-1)
    uu = jnp.sum(x * x / s0, axis=-1)
    logdet = jnp.sum(jnp.log(st), axis=-1) - jnp.sum(jnp.log(s0), axis=-1)
    o_ref[...] = 0.5 * (trace + uu - D - logdet)


def _tc_math(emb_g, cov_g, bB=128, interpret=False):
    B, S, dpad = emb_g.shape
    return pl.pallas_call(
        _math_body,
        grid=(B // bB,),
        in_specs=[pl.BlockSpec((bB, S, dpad), lambda i: (i, 0, 0)),
                  pl.BlockSpec((bB, S, D), lambda i: (i, 0, 0))],
        out_specs=pl.BlockSpec((bB, S - 1), lambda i: (i, 0)),
        out_shape=jax.ShapeDtypeStruct((B, S - 1), jnp.float32),
        interpret=interpret,
    )(emb_g, cov_g)


def kernel(idx, embedding, covariance):
    B, S = idx.shape
    idx_flat = idx.reshape(-1)
    emb_pad = jnp.pad(embedding, ((0, 0), (0, DPAD - DP1)))
    emb_g, cov_g = _sc_gather(idx_flat, emb_pad, covariance)
    return _tc_math(emb_g.reshape(B, S, DPAD), cov_g.reshape(B, S, D))
